# baseline (speedup 1.0000x reference)
import functools

import jax
import jax.numpy as jnp
from jax import lax
from jax.experimental import pallas as pl
from jax.experimental.pallas import tpu as pltpu
from jax.experimental.pallas import tpu_sc as plsc

N_NODES = 10000
N_EDGES = 320000
BATCH = 4096
DIM = 128

NT = 16
EC = N_EDGES // NT
CB = 624
CH = 640
NV_N = N_NODES // 16
NV_E = EC // 16
NV_C = CH // 16
UB = BATCH // NT


def _prep_body(itab_ref, w1_ref, w2_ref, wl_ref, b1_ref, b2_ref,
               g0_ref, svec_ref):
    wi = wl_ref[128:256, :]
    w2i = jnp.dot(w2_ref[...], wi, preferred_element_type=jnp.float32)
    w12 = jnp.dot(w1_ref[...], w2i, preferred_element_type=jnp.float32)
    g0_ref[...] = jnp.dot(itab_ref[...], w12,
                          preferred_element_type=jnp.float32)
    s1 = jnp.dot(b1_ref[...], w2i, preferred_element_type=jnp.float32)
    s2 = jnp.dot(b2_ref[...], wi, preferred_element_type=jnp.float32)
    col = lax.broadcasted_iota(jnp.int32, (1, 32), 1)
    svec_ref[...] = jnp.where(col < 16, s1[0, 0], s2[0, 0])


def _prep(i_table, W1, W2, Wl, b1_2d, b2_2d):
    return pl.pallas_call(
        _prep_body,
        out_shape=[
            jax.ShapeDtypeStruct((N_NODES, 1), jnp.float32),
            jax.ShapeDtypeStruct((1, 32), jnp.float32),
        ],
    )(i_table, W1, W2, Wl, b1_2d, b2_2d)


def _sc_body(src_hbm, dst_hbm, x_hbm, g0_hbm, svec_hbm, user_hbm, utab_hbm,
             v_out, uemb_out,
             b_src, b_dst, b_g, b_dinv, b_acc, b_x, b_chunk, b_chunk2,
             b_svec, b_uidx, b_urows, sh_all, sh_g, sem):
    core = lax.axis_index("c")
    tid = lax.axis_index("s")

    @pl.when(core == 1)
    def _():
        pltpu.sync_copy(user_hbm.at[tid], b_uidx)
        for j in range(UB // 128):
            pltpu.async_copy(utab_hbm.at[b_uidx.at[j]],
                             b_urows.at[pl.ds(j * 128, 128)], sem).wait()
        pltpu.sync_copy(b_urows, uemb_out.at[pl.ds(tid * UB, UB)])

    def zero_acc():
        @plsc.parallel_loop(0, NV_N, unroll=8)
        def _(i):
            b_acc[pl.ds(i * 16, 16)] = jnp.zeros((16,), jnp.float32)

    def my_row_off(t):
        return pl.multiple_of(t * N_NODES + tid * CB, 8)

    def combine_my_chunk():
        pltpu.sync_copy(sh_all.at[pl.ds(my_row_off(0), CH)], b_chunk)
        for t in range(1, NT):
            pltpu.sync_copy(sh_all.at[pl.ds(my_row_off(t), CH)], b_chunk2)

            def addrow(i, c):
                b_chunk[pl.ds(i * 16, 16)] = (
                    b_chunk[pl.ds(i * 16, 16)] + b_chunk2[pl.ds(i * 16, 16)])
                return c
            lax.fori_loop(0, NV_C, addrow, 0)

    @pl.when(core == 0)
    def _():
        pltpu.sync_copy(src_hbm.at[pl.ds(tid * EC, EC)], b_src)
        pltpu.sync_copy(dst_hbm.at[pl.ds(tid * EC, EC)], b_dst)
        pltpu.sync_copy(svec_hbm, b_svec)
        zero_acc()
        ones = jnp.full((16,), 1.0, jnp.float32)

        def deg_step(i, c):
            dv = b_dst[pl.ds(i * 16, 16)]
            plsc.addupdate_scatter(b_acc, [dv], ones)
            return c
        lax.fori_loop(0, NV_E, deg_step, 0, unroll=8)
        pltpu.sync_copy(
            b_acc,
            sh_all.at[pl.ds(pl.multiple_of(tid * N_NODES, 8), N_NODES)])
    plsc.subcore_barrier()

    @pl.when(core == 0)
    def _():
        combine_my_chunk()

        def newton(i, c):
            d = b_chunk[pl.ds(i * 16, 16)] + 1.0
            bi = plsc.bitcast(d, jnp.int32)
            bi = 0x5F3759DF - lax.shift_right_arithmetic(bi, 1)
            y = plsc.bitcast(bi, jnp.float32)
            y = y * (1.5 - 0.5 * d * y * y)
            y = y * (1.5 - 0.5 * d * y * y)
            y = y * (1.5 - 0.5 * d * y * y)
            b_chunk[pl.ds(i * 16, 16)] = y
            return c
        lax.fori_loop(0, NV_C, newton, 0)
        pltpu.sync_copy(b_chunk, sh_g.at[pl.ds(tid * CB, CH)])
    plsc.subcore_barrier()
    @pl.when(core == 0)
    def _():
        pltpu.sync_copy(sh_g, b_dinv)
    plsc.subcore_barrier()

    @pl.when(core == 0)
    def _():
        pltpu.sync_copy(g0_hbm, b_g)
        pltpu.sync_copy(x_hbm.at[pl.ds(tid * CB, CH)], b_x)

        @plsc.parallel_loop(0, NV_C, unroll=4)
        def _(i):
            xi = b_x[pl.ds(i * 16, 16)]
            d16 = b_dinv[pl.ds(tid * CB + i * 16, 16)]
            b_chunk[pl.ds(i * 16, 16)] = plsc.load_gather(b_g, [xi]) * d16
        pltpu.sync_copy(b_chunk, sh_g.at[pl.ds(tid * CB, CH)])
    plsc.subcore_barrier()

    def graph_round(s_off, write_hbm):
        @pl.when(core == 0)
        def _():
            pltpu.sync_copy(sh_g, b_g)
            zero_acc()

            def edge_step(i, c):
                sv = b_src[pl.ds(i * 16, 16)]
                dv = b_dst[pl.ds(i * 16, 16)]
                ps = plsc.load_gather(b_g, [sv])
                plsc.addupdate_scatter(b_acc, [dv], ps)
                return c
            lax.fori_loop(0, NV_E, edge_step, 0, unroll=8)
            pltpu.sync_copy(
                b_acc,
                sh_all.at[pl.ds(pl.multiple_of(tid * N_NODES, 8), N_NODES)])
        plsc.subcore_barrier()

        @pl.when(core == 0)
        def _():
            combine_my_chunk()
            sv = b_svec[pl.ds(s_off, 16)]

            @plsc.parallel_loop(0, NV_C, unroll=4)
            def _(i):
                p16 = b_g[pl.ds(tid * CB + i * 16, 16)]
                d16 = b_dinv[pl.ds(tid * CB + i * 16, 16)]
                g_new = (b_chunk[pl.ds(i * 16, 16)] + p16) * d16 + sv
                if write_hbm:
                    b_chunk[pl.ds(i * 16, 16)] = g_new
                else:
                    b_chunk[pl.ds(i * 16, 16)] = g_new * d16
            if write_hbm:
                pltpu.sync_copy(b_chunk, v_out.at[pl.ds(tid * CB, CH)])
            else:
                pltpu.sync_copy(b_chunk, sh_g.at[pl.ds(tid * CB, CH)])
        if not write_hbm:
            plsc.subcore_barrier()

    graph_round(0, False)
    graph_round(16, True)


def _sc_graph(src, dst, x, g0, svec, user_r, u_table):
    mesh = plsc.VectorSubcoreMesh(core_axis_name="c", subcore_axis_name="s")
    f = pl.kernel(
        _sc_body,
        out_type=[
            jax.ShapeDtypeStruct((N_NODES,), jnp.float32),
            jax.ShapeDtypeStruct((BATCH, DIM), jnp.float32),
        ],
        mesh=mesh,
        scratch_types=[
            pltpu.VMEM((EC,), jnp.int32),
            pltpu.VMEM((EC,), jnp.int32),
            pltpu.VMEM((N_NODES,), jnp.float32),
            pltpu.VMEM((N_NODES,), jnp.float32),
            pltpu.VMEM((N_NODES,), jnp.float32),
            pltpu.VMEM((CH,), jnp.int32),
            pltpu.VMEM((CH,), jnp.float32),
            pltpu.VMEM((CH,), jnp.float32),
            pltpu.VMEM((32,), jnp.float32),
            pltpu.VMEM((UB // 128, 128), jnp.int32),
            pltpu.VMEM((UB, DIM), jnp.float32),
            pltpu.VMEM_SHARED((NT * N_NODES,), jnp.float32),
            pltpu.VMEM_SHARED((N_NODES,), jnp.float32),
            pltpu.SemaphoreType.DMA,
        ],
        compiler_params=pltpu.CompilerParams(needs_layout_passes=False),
    )
    return f(src, dst, x, g0, svec, user_r, u_table)


_BM = 256


def _final_body(item_ref, v_ref, uemb_ref, wu_ref, bl_ref, out_ref):
    acc = jnp.sum(item_ref[...] * v_ref[...], axis=1, keepdims=True)
    accu = jnp.sum(uemb_ref[...] * wu_ref[...], axis=1, keepdims=True)
    out_ref[...] = jax.nn.sigmoid(acc + accu + bl_ref[0, 0])


def _final(item, v2d, uemb, wu2d, bl2d):
    n = item.shape[1]
    return pl.pallas_call(
        _final_body,
        grid=(BATCH // _BM,),
        in_specs=[
            pl.BlockSpec((_BM, n), lambda i: (i, 0)),
            pl.BlockSpec((1, n), lambda i: (0, 0)),
            pl.BlockSpec((_BM, DIM), lambda i: (i, 0)),
            pl.BlockSpec((1, DIM), lambda i: (0, 0)),
            pl.BlockSpec((1, 1), lambda i: (0, 0)),
        ],
        out_specs=pl.BlockSpec((_BM, 1), lambda i: (i, 0)),
        out_shape=jax.ShapeDtypeStruct((BATCH, 1), jnp.float32),
        compiler_params=pltpu.CompilerParams(
            dimension_semantics=("parallel",)),
    )(item, v2d, uemb, wu2d, bl2d)


def kernel(user, item, x, edge_index, u_table, i_table, W1, b1, W2, b2,
           Wl, bl):
    g0_2d, svec_2d = _prep(i_table, W1, W2, Wl,
                           b1.reshape(1, DIM), b2.reshape(1, DIM))
    src = edge_index[0].astype(jnp.int32)
    dst = edge_index[1].astype(jnp.int32)
    user_r = user.astype(jnp.int32).reshape(NT, UB // 128, 128)
    v, uemb = _sc_graph(src, dst, x.astype(jnp.int32),
                        g0_2d.reshape(N_NODES), svec_2d.reshape(32),
                        user_r, u_table)
    wu2d = Wl[:DIM].reshape(1, DIM)
    return _final(item, v.reshape(1, N_NODES), uemb, wu2d,
                  bl.reshape(1, 1))


# speedup vs baseline: 1.5474x; 1.5474x over previous
import functools

import jax
import jax.numpy as jnp
from jax import lax
from jax.experimental import pallas as pl
from jax.experimental.pallas import tpu as pltpu
from jax.experimental.pallas import tpu_sc as plsc

N_NODES = 10000
N_EDGES = 320000
BATCH = 4096
DIM = 128

NT = 16
EC = N_EDGES // NT
CB = 624
CH = 640
NV_N = N_NODES // 16
NV_E = EC // 16
NV_C = CH // 16
UB = BATCH // NT


def _prep_body(itab_ref, w1_ref, w2_ref, wl_ref, b1_ref, b2_ref,
               g0_ref, svec_ref):
    wi = wl_ref[128:256, :]
    w2i = jnp.dot(w2_ref[...], wi, preferred_element_type=jnp.float32)
    w12 = jnp.dot(w1_ref[...], w2i, preferred_element_type=jnp.float32)
    g0_ref[...] = jnp.dot(itab_ref[...], w12,
                          preferred_element_type=jnp.float32)
    s1 = jnp.dot(b1_ref[...], w2i, preferred_element_type=jnp.float32)
    s2 = jnp.dot(b2_ref[...], wi, preferred_element_type=jnp.float32)
    col = lax.broadcasted_iota(jnp.int32, (1, 32), 1)
    svec_ref[...] = jnp.where(col < 16, s1[0, 0], s2[0, 0])


def _prep(i_table, W1, W2, Wl, b1_2d, b2_2d):
    return pl.pallas_call(
        _prep_body,
        out_shape=[
            jax.ShapeDtypeStruct((N_NODES, 1), jnp.float32),
            jax.ShapeDtypeStruct((1, 32), jnp.float32),
        ],
    )(i_table, W1, W2, Wl, b1_2d, b2_2d)


def _sc_body(src_hbm, dst_hbm, x_hbm, g0_hbm, svec_hbm, user_hbm, utab_hbm,
             v_out, uemb_out,
             b_src, b_dst, b_g, b_dinv, b_acc, b_x, b_chunk, b_chunk2,
             b_svec, b_uidx, b_urows, sh_all, sh_g, sem):
    core = lax.axis_index("c")
    tid = lax.axis_index("s")

    @pl.when(core == 1)
    def _():
        pltpu.sync_copy(user_hbm.at[tid], b_uidx)
        for j in range(UB // 128):
            pltpu.async_copy(utab_hbm.at[b_uidx.at[j]],
                             b_urows.at[pl.ds(j * 128, 128)], sem).wait()
        pltpu.sync_copy(b_urows, uemb_out.at[pl.ds(tid * UB, UB)])

    def zero_acc():
        @plsc.parallel_loop(0, NV_N, unroll=8)
        def _(i):
            b_acc[pl.ds(i * 16, 16)] = jnp.zeros((16,), jnp.float32)

    def my_row_off(t):
        return pl.multiple_of(t * N_NODES + tid * CB, 8)

    def combine_my_chunk():
        pltpu.sync_copy(sh_all.at[pl.ds(my_row_off(0), CH)], b_chunk)
        for t in range(1, NT):
            pltpu.sync_copy(sh_all.at[pl.ds(my_row_off(t), CH)], b_chunk2)

            def addrow(i, c):
                b_chunk[pl.ds(i * 16, 16)] = (
                    b_chunk[pl.ds(i * 16, 16)] + b_chunk2[pl.ds(i * 16, 16)])
                return c
            lax.fori_loop(0, NV_C, addrow, 0)

    @pl.when(core == 0)
    def _():
        pltpu.sync_copy(src_hbm.at[pl.ds(tid * EC, EC)], b_src)
        pltpu.sync_copy(dst_hbm.at[pl.ds(tid * EC, EC)], b_dst)
        pltpu.sync_copy(svec_hbm, b_svec)
        zero_acc()
        ones = jnp.full((16,), 1.0, jnp.float32)

        def deg_step(i, c):
            dv = b_dst[pl.ds(i * 16, 16)]
            plsc.addupdate_scatter(b_acc, [dv], ones)
            return c
        lax.fori_loop(0, NV_E, deg_step, 0, unroll=8)
        pltpu.sync_copy(
            b_acc,
            sh_all.at[pl.ds(pl.multiple_of(tid * N_NODES, 8), N_NODES)])
    plsc.subcore_barrier()

    @pl.when(core == 0)
    def _():
        combine_my_chunk()

        def newton(i, c):
            d = b_chunk[pl.ds(i * 16, 16)] + 1.0
            bi = plsc.bitcast(d, jnp.int32)
            bi = 0x5F3759DF - lax.shift_right_arithmetic(bi, 1)
            y = plsc.bitcast(bi, jnp.float32)
            y = y * (1.5 - 0.5 * d * y * y)
            y = y * (1.5 - 0.5 * d * y * y)
            y = y * (1.5 - 0.5 * d * y * y)
            b_chunk[pl.ds(i * 16, 16)] = y
            return c
        lax.fori_loop(0, NV_C, newton, 0)
        pltpu.sync_copy(b_chunk, sh_g.at[pl.ds(tid * CB, CH)])
    plsc.subcore_barrier()
    @pl.when(core == 0)
    def _():
        pltpu.sync_copy(sh_g, b_dinv)
    plsc.subcore_barrier()

    @pl.when(core == 0)
    def _():
        pltpu.sync_copy(g0_hbm, b_g)
        pltpu.sync_copy(x_hbm.at[pl.ds(tid * CB, CH)], b_x)

        @plsc.parallel_loop(0, NV_C, unroll=4)
        def _(i):
            xi = b_x[pl.ds(i * 16, 16)]
            d16 = b_dinv[pl.ds(tid * CB + i * 16, 16)]
            b_chunk[pl.ds(i * 16, 16)] = plsc.load_gather(b_g, [xi]) * d16
        pltpu.sync_copy(b_chunk, sh_g.at[pl.ds(tid * CB, CH)])
    plsc.subcore_barrier()

    def graph_round(s_off, write_hbm):
        @pl.when(core == 0)
        def _():
            pltpu.sync_copy(sh_g, b_g)
            zero_acc()

            def edge_step(i, c):
                sv = b_src[pl.ds(i * 16, 16)]
                dv = b_dst[pl.ds(i * 16, 16)]
                ps = plsc.load_gather(b_g, [sv])
                plsc.addupdate_scatter(b_acc, [dv], ps)
                return c
            lax.fori_loop(0, NV_E, edge_step, 0, unroll=8)
            pltpu.sync_copy(
                b_acc,
                sh_all.at[pl.ds(pl.multiple_of(tid * N_NODES, 8), N_NODES)])
        plsc.subcore_barrier()

        @pl.when(core == 0)
        def _():
            combine_my_chunk()
            sv = b_svec[pl.ds(s_off, 16)]

            @plsc.parallel_loop(0, NV_C, unroll=4)
            def _(i):
                p16 = b_g[pl.ds(tid * CB + i * 16, 16)]
                d16 = b_dinv[pl.ds(tid * CB + i * 16, 16)]
                g_new = (b_chunk[pl.ds(i * 16, 16)] + p16) * d16 + sv
                if write_hbm:
                    b_chunk[pl.ds(i * 16, 16)] = g_new
                else:
                    b_chunk[pl.ds(i * 16, 16)] = g_new * d16
            if write_hbm:
                pltpu.sync_copy(b_chunk, v_out.at[pl.ds(tid * CB, CH)])
            else:
                pltpu.sync_copy(b_chunk, sh_g.at[pl.ds(tid * CB, CH)])
        if not write_hbm:
            plsc.subcore_barrier()

    graph_round(0, False)
    graph_round(16, True)


def _sc_graph(src, dst, x, g0, svec, user_r, u_table):
    mesh = plsc.VectorSubcoreMesh(core_axis_name="c", subcore_axis_name="s")
    f = pl.kernel(
        _sc_body,
        out_type=[
            jax.ShapeDtypeStruct((N_NODES,), jnp.float32),
            jax.ShapeDtypeStruct((BATCH, DIM), jnp.float32),
        ],
        mesh=mesh,
        scratch_types=[
            pltpu.VMEM((EC,), jnp.int32),
            pltpu.VMEM((EC,), jnp.int32),
            pltpu.VMEM((N_NODES,), jnp.float32),
            pltpu.VMEM((N_NODES,), jnp.float32),
            pltpu.VMEM((N_NODES,), jnp.float32),
            pltpu.VMEM((CH,), jnp.int32),
            pltpu.VMEM((CH,), jnp.float32),
            pltpu.VMEM((CH,), jnp.float32),
            pltpu.VMEM((32,), jnp.float32),
            pltpu.VMEM((UB // 128, 128), jnp.int32),
            pltpu.VMEM((UB, DIM), jnp.float32),
            pltpu.VMEM_SHARED((NT * N_NODES,), jnp.float32),
            pltpu.VMEM_SHARED((N_NODES,), jnp.float32),
            pltpu.SemaphoreType.DMA,
        ],
        compiler_params=pltpu.CompilerParams(needs_layout_passes=False),
    )
    return f(src, dst, x, g0, svec, user_r, u_table)


_BK = 1000
_GK = N_NODES // _BK


def _final_body(itemT_ref, v_ref, uemb_ref, wu_ref, bl_ref, out_ref):
    k = pl.program_id(0)
    part = jnp.sum(itemT_ref[...] * v_ref[...], axis=0, keepdims=True)

    @pl.when(k == 0)
    def _():
        su = lax.dot_general(wu_ref[...], uemb_ref[...],
                             (((1,), (1,)), ((), ())),
                             preferred_element_type=jnp.float32)
        out_ref[...] = part + su + bl_ref[0, 0]

    @pl.when(k > 0)
    def _():
        out_ref[...] = out_ref[...] + part

    @pl.when(k == _GK - 1)
    def _():
        out_ref[...] = jax.nn.sigmoid(out_ref[...])


def _final(itemT, v2d, uemb, wu2d, bl2d):
    return pl.pallas_call(
        _final_body,
        grid=(_GK,),
        in_specs=[
            pl.BlockSpec((_BK, BATCH), lambda k: (k, 0)),
            pl.BlockSpec((_BK, 1), lambda k: (k, 0)),
            pl.BlockSpec((BATCH, DIM), lambda k: (0, 0)),
            pl.BlockSpec((1, DIM), lambda k: (0, 0)),
            pl.BlockSpec((1, 1), lambda k: (0, 0)),
        ],
        out_specs=pl.BlockSpec((1, BATCH), lambda k: (0, 0)),
        out_shape=jax.ShapeDtypeStruct((1, BATCH), jnp.float32),
        compiler_params=pltpu.CompilerParams(
            dimension_semantics=("arbitrary",),
            vmem_limit_bytes=50 * 1024 * 1024),
    )(itemT, v2d, uemb, wu2d, bl2d)


def kernel(user, item, x, edge_index, u_table, i_table, W1, b1, W2, b2,
           Wl, bl):
    g0_2d, svec_2d = _prep(i_table, W1, W2, Wl,
                           b1.reshape(1, DIM), b2.reshape(1, DIM))
    src = edge_index[0].astype(jnp.int32)
    dst = edge_index[1].astype(jnp.int32)
    user_r = user.astype(jnp.int32).reshape(NT, UB // 128, 128)
    v, uemb = _sc_graph(src, dst, x.astype(jnp.int32),
                        g0_2d.reshape(N_NODES), svec_2d.reshape(32),
                        user_r, u_table)
    wu2d = Wl[:DIM].reshape(1, DIM)
    out_row = _final(item.T, v.reshape(N_NODES, 1), uemb, wu2d,
                     bl.reshape(1, 1))
    return out_row.reshape(BATCH, 1)


# speedup vs baseline: 1.6834x; 1.0879x over previous
import functools

import jax
import jax.numpy as jnp
from jax import lax
from jax.experimental import pallas as pl
from jax.experimental.pallas import tpu as pltpu
from jax.experimental.pallas import tpu_sc as plsc

N_NODES = 10000
N_EDGES = 320000
BATCH = 4096
DIM = 128

NT = 16
EC = N_EDGES // NT
CB = 624
CH = 640
NV_N = N_NODES // 16
NV_E = EC // 16
NV_C = CH // 16
UB = BATCH // NT


def _contract1(row, mat):
    return lax.dot_general(row, mat, (((1,), (1,)), ((), ())),
                           preferred_element_type=jnp.float32)


def _prep_body(itab_ref, w1_ref, w2_ref, wlT_ref, b1_ref, b2_ref,
               g0_ref, svec_ref):
    wiT = wlT_ref[:, 128:256]
    w2iT = _contract1(wiT, w2_ref[...])
    w12T = _contract1(w2iT, w1_ref[...])
    g0_ref[...] = _contract1(w12T, itab_ref[...]).reshape(N_NODES)
    s1 = jnp.sum(b1_ref[...] * w2iT)
    s2 = jnp.sum(b2_ref[...] * wiT)
    col = lax.broadcasted_iota(jnp.int32, (1, 32), 1)
    svec_ref[...] = jnp.where(col < 16, s1, s2).reshape(32)


def _prep(i_table, W1, W2, WlT, b1_2d, b2_2d):
    return pl.pallas_call(
        _prep_body,
        out_shape=[
            jax.ShapeDtypeStruct((N_NODES,), jnp.float32),
            jax.ShapeDtypeStruct((32,), jnp.float32),
        ],
    )(i_table, W1, W2, WlT, b1_2d, b2_2d)


def _sc_body(src_hbm, dst_hbm, x_hbm, g0_hbm, svec_hbm, user_hbm, utab_hbm,
             v_out, uemb_out,
             b_src, b_dst, b_g, b_dinv, b_acc, b_x, b_chunk, b_chunk2,
             b_svec, b_uidx, b_urows, sh_all, sh_g, sem):
    core = lax.axis_index("c")
    tid = lax.axis_index("s")

    @pl.when(core == 1)
    def _():
        pltpu.sync_copy(user_hbm.at[tid], b_uidx)
        for j in range(UB // 128):
            pltpu.async_copy(utab_hbm.at[b_uidx.at[j]],
                             b_urows.at[pl.ds(j * 128, 128)], sem).wait()
        pltpu.sync_copy(b_urows, uemb_out.at[pl.ds(tid * UB, UB)])

    def zero_acc():
        @plsc.parallel_loop(0, NV_N, unroll=8)
        def _(i):
            b_acc[pl.ds(i * 16, 16)] = jnp.zeros((16,), jnp.float32)

    def my_row_off(t):
        return pl.multiple_of(t * N_NODES + tid * CB, 8)

    def combine_my_chunk():
        pltpu.sync_copy(sh_all.at[pl.ds(my_row_off(0), CH)], b_chunk)
        for t in range(1, NT):
            pltpu.sync_copy(sh_all.at[pl.ds(my_row_off(t), CH)], b_chunk2)

            def addrow(i, c):
                b_chunk[pl.ds(i * 16, 16)] = (
                    b_chunk[pl.ds(i * 16, 16)] + b_chunk2[pl.ds(i * 16, 16)])
                return c
            lax.fori_loop(0, NV_C, addrow, 0)

    @pl.when(core == 0)
    def _():
        pltpu.sync_copy(src_hbm.at[pl.ds(tid * EC, EC)], b_src)
        pltpu.sync_copy(dst_hbm.at[pl.ds(tid * EC, EC)], b_dst)
        pltpu.sync_copy(svec_hbm, b_svec)
        zero_acc()
        ones = jnp.full((16,), 1.0, jnp.float32)

        def deg_step(i, c):
            dv = b_dst[pl.ds(i * 16, 16)]
            plsc.addupdate_scatter(b_acc, [dv], ones)
            return c
        lax.fori_loop(0, NV_E, deg_step, 0, unroll=8)
        pltpu.sync_copy(
            b_acc,
            sh_all.at[pl.ds(pl.multiple_of(tid * N_NODES, 8), N_NODES)])
    plsc.subcore_barrier()

    @pl.when(core == 0)
    def _():
        combine_my_chunk()

        def newton(i, c):
            d = b_chunk[pl.ds(i * 16, 16)] + 1.0
            bi = plsc.bitcast(d, jnp.int32)
            bi = 0x5F3759DF - lax.shift_right_arithmetic(bi, 1)
            y = plsc.bitcast(bi, jnp.float32)
            y = y * (1.5 - 0.5 * d * y * y)
            y = y * (1.5 - 0.5 * d * y * y)
            y = y * (1.5 - 0.5 * d * y * y)
            b_chunk[pl.ds(i * 16, 16)] = y
            return c
        lax.fori_loop(0, NV_C, newton, 0)
        pltpu.sync_copy(b_chunk, sh_g.at[pl.ds(tid * CB, CH)])
    plsc.subcore_barrier()
    @pl.when(core == 0)
    def _():
        pltpu.sync_copy(sh_g, b_dinv)
    plsc.subcore_barrier()

    @pl.when(core == 0)
    def _():
        pltpu.sync_copy(g0_hbm, b_g)
        pltpu.sync_copy(x_hbm.at[pl.ds(tid * CB, CH)], b_x)

        @plsc.parallel_loop(0, NV_C, unroll=4)
        def _(i):
            xi = b_x[pl.ds(i * 16, 16)]
            d16 = b_dinv[pl.ds(tid * CB + i * 16, 16)]
            b_chunk[pl.ds(i * 16, 16)] = plsc.load_gather(b_g, [xi]) * d16
        pltpu.sync_copy(b_chunk, sh_g.at[pl.ds(tid * CB, CH)])
    plsc.subcore_barrier()

    def graph_round(s_off, write_hbm):
        @pl.when(core == 0)
        def _():
            pltpu.sync_copy(sh_g, b_g)
            zero_acc()

            def edge_step(i, c):
                sv = b_src[pl.ds(i * 16, 16)]
                dv = b_dst[pl.ds(i * 16, 16)]
                ps = plsc.load_gather(b_g, [sv])
                plsc.addupdate_scatter(b_acc, [dv], ps)
                return c
            lax.fori_loop(0, NV_E, edge_step, 0, unroll=8)
            pltpu.sync_copy(
                b_acc,
                sh_all.at[pl.ds(pl.multiple_of(tid * N_NODES, 8), N_NODES)])
        plsc.subcore_barrier()

        @pl.when(core == 0)
        def _():
            combine_my_chunk()
            sv = b_svec[pl.ds(s_off, 16)]

            @plsc.parallel_loop(0, NV_C, unroll=4)
            def _(i):
                p16 = b_g[pl.ds(tid * CB + i * 16, 16)]
                d16 = b_dinv[pl.ds(tid * CB + i * 16, 16)]
                g_new = (b_chunk[pl.ds(i * 16, 16)] + p16) * d16 + sv
                if write_hbm:
                    b_chunk[pl.ds(i * 16, 16)] = g_new
                else:
                    b_chunk[pl.ds(i * 16, 16)] = g_new * d16
            if write_hbm:
                pltpu.sync_copy(b_chunk, v_out.at[pl.ds(tid * CB, CH)])
            else:
                pltpu.sync_copy(b_chunk, sh_g.at[pl.ds(tid * CB, CH)])
        if not write_hbm:
            plsc.subcore_barrier()

    graph_round(0, False)
    graph_round(16, True)


def _sc_graph(src, dst, x, g0, svec, user_r, u_table):
    mesh = plsc.VectorSubcoreMesh(core_axis_name="c", subcore_axis_name="s")
    f = pl.kernel(
        _sc_body,
        out_type=[
            jax.ShapeDtypeStruct((N_NODES,), jnp.float32),
            jax.ShapeDtypeStruct((BATCH, DIM), jnp.float32),
        ],
        mesh=mesh,
        scratch_types=[
            pltpu.VMEM((EC,), jnp.int32),
            pltpu.VMEM((EC,), jnp.int32),
            pltpu.VMEM((N_NODES,), jnp.float32),
            pltpu.VMEM((N_NODES,), jnp.float32),
            pltpu.VMEM((N_NODES,), jnp.float32),
            pltpu.VMEM((CH,), jnp.int32),
            pltpu.VMEM((CH,), jnp.float32),
            pltpu.VMEM((CH,), jnp.float32),
            pltpu.VMEM((32,), jnp.float32),
            pltpu.VMEM((UB // 128, 128), jnp.int32),
            pltpu.VMEM((UB, DIM), jnp.float32),
            pltpu.VMEM_SHARED((NT * N_NODES,), jnp.float32),
            pltpu.VMEM_SHARED((N_NODES,), jnp.float32),
            pltpu.SemaphoreType.DMA,
        ],
        compiler_params=pltpu.CompilerParams(needs_layout_passes=False),
    )
    return f(src, dst, x, g0, svec, user_r, u_table)


_BN = 512
_GN = BATCH // _BN


def _final_body(itemT_ref, v_ref, uemb_ref, wu_ref, bl_ref, out_ref):
    vrow = v_ref[...].reshape(1, N_NODES)
    part = jnp.dot(vrow, itemT_ref[...], preferred_element_type=jnp.float32)
    su = lax.dot_general(wu_ref[...], uemb_ref[...],
                         (((1,), (1,)), ((), ())),
                         preferred_element_type=jnp.float32)
    out_ref[...] = jax.nn.sigmoid(part + su + bl_ref[0, 0])


def _final(itemT, v, uemb, wu2d, bl2d):
    return pl.pallas_call(
        _final_body,
        grid=(_GN,),
        in_specs=[
            pl.BlockSpec((N_NODES, _BN), lambda n: (0, n)),
            pl.BlockSpec((N_NODES,), lambda n: (0,)),
            pl.BlockSpec((_BN, DIM), lambda n: (n, 0)),
            pl.BlockSpec((1, DIM), lambda n: (0, 0)),
            pl.BlockSpec((1, 1), lambda n: (0, 0)),
        ],
        out_specs=pl.BlockSpec((1, _BN), lambda n: (0, n)),
        out_shape=jax.ShapeDtypeStruct((1, BATCH), jnp.float32),
        compiler_params=pltpu.CompilerParams(
            dimension_semantics=("arbitrary",),
            vmem_limit_bytes=50 * 1024 * 1024),
    )(itemT, v, uemb, wu2d, bl2d)


def kernel(user, item, x, edge_index, u_table, i_table, W1, b1, W2, b2,
           Wl, bl):
    WlT = Wl.reshape(1, 2 * DIM)
    g0, svec = _prep(i_table, W1, W2, WlT,
                     b1.reshape(1, DIM), b2.reshape(1, DIM))
    src = edge_index[0].astype(jnp.int32)
    dst = edge_index[1].astype(jnp.int32)
    user_r = user.astype(jnp.int32).reshape(NT, UB // 128, 128)
    v, uemb = _sc_graph(src, dst, x.astype(jnp.int32), g0, svec,
                        user_r, u_table)
    wu2d = WlT[:, :DIM]
    out_row = _final(item.T, v, uemb, wu2d, bl.reshape(1, 1))
    return out_row.reshape(BATCH, 1)


# speedup vs baseline: 1.6880x; 1.0027x over previous
import functools

import jax
import jax.numpy as jnp
from jax import lax
from jax.experimental import pallas as pl
from jax.experimental.pallas import tpu as pltpu
from jax.experimental.pallas import tpu_sc as plsc

N_NODES = 10000
N_EDGES = 320000
BATCH = 4096
DIM = 128

NT = 16
EC = N_EDGES // NT
CB = 624
CH = 640
NV_N = N_NODES // 16
NV_E = EC // 16
NV_C = CH // 16
UB = BATCH // NT


def _contract1(row, mat):
    return lax.dot_general(row, mat, (((1,), (1,)), ((), ())),
                           preferred_element_type=jnp.float32)


def _prep_body(itab_ref, w1_ref, w2_ref, wlT_ref, b1_ref, b2_ref,
               g0_ref, svec_ref):
    wiT = wlT_ref[:, 128:256]
    w2iT = _contract1(wiT, w2_ref[...])
    w12T = _contract1(w2iT, w1_ref[...])
    g0_ref[...] = _contract1(w12T, itab_ref[...]).reshape(N_NODES)
    s1 = jnp.sum(b1_ref[...] * w2iT)
    s2 = jnp.sum(b2_ref[...] * wiT)
    col = lax.broadcasted_iota(jnp.int32, (1, 32), 1)
    svec_ref[...] = jnp.where(col < 16, s1, s2).reshape(32)


def _prep(i_table, W1, W2, WlT, b1_2d, b2_2d):
    return pl.pallas_call(
        _prep_body,
        out_shape=[
            jax.ShapeDtypeStruct((N_NODES,), jnp.float32),
            jax.ShapeDtypeStruct((32,), jnp.float32),
        ],
    )(i_table, W1, W2, WlT, b1_2d, b2_2d)


def _sc_body(src_hbm, dst_hbm, x_hbm, g0_hbm, svec_hbm, user_hbm, utab_hbm,
             v_out, uemb_out,
             b_src, b_dst, b_g, b_dinv, b_acc, b_x, b_chunk, b_chunk2,
             b_svec, b_uidx, b_urows, sh_all, sh_g, sem):
    core = lax.axis_index("c")
    tid = lax.axis_index("s")

    @pl.when(core == 1)
    def _():
        pltpu.sync_copy(user_hbm.at[tid], b_uidx)
        for j in range(UB // 128):
            pltpu.async_copy(utab_hbm.at[b_uidx.at[j]],
                             b_urows.at[pl.ds(j * 128, 128)], sem).wait()
        pltpu.sync_copy(b_urows, uemb_out.at[pl.ds(tid * UB, UB)])

    def zero_acc():
        @plsc.parallel_loop(0, NV_N, unroll=8)
        def _(i):
            b_acc[pl.ds(i * 16, 16)] = jnp.zeros((16,), jnp.float32)

    def my_row_off(t):
        return pl.multiple_of(t * N_NODES + tid * CB, 8)

    def combine_my_chunk():
        pltpu.sync_copy(sh_all.at[pl.ds(my_row_off(0), CH)], b_chunk)
        for t in range(1, NT):
            pltpu.sync_copy(sh_all.at[pl.ds(my_row_off(t), CH)], b_chunk2)

            def addrow(i, c):
                b_chunk[pl.ds(i * 16, 16)] = (
                    b_chunk[pl.ds(i * 16, 16)] + b_chunk2[pl.ds(i * 16, 16)])
                return c
            lax.fori_loop(0, NV_C, addrow, 0)

    @pl.when(core == 0)
    def _():
        pltpu.sync_copy(src_hbm.at[pl.ds(tid * EC, EC)], b_src)
        pltpu.sync_copy(dst_hbm.at[pl.ds(tid * EC, EC)], b_dst)
        pltpu.sync_copy(svec_hbm, b_svec)
        zero_acc()
        ones = jnp.full((16,), 1.0, jnp.float32)

        def deg_step(i, c):
            dv = b_dst[pl.ds(i * 16, 16)]
            plsc.addupdate_scatter(b_acc, [dv], ones)
            return c
        lax.fori_loop(0, NV_E, deg_step, 0, unroll=16)
        pltpu.sync_copy(
            b_acc,
            sh_all.at[pl.ds(pl.multiple_of(tid * N_NODES, 8), N_NODES)])
    plsc.subcore_barrier()

    @pl.when(core == 0)
    def _():
        combine_my_chunk()

        def newton(i, c):
            d = b_chunk[pl.ds(i * 16, 16)] + 1.0
            bi = plsc.bitcast(d, jnp.int32)
            bi = 0x5F3759DF - lax.shift_right_arithmetic(bi, 1)
            y = plsc.bitcast(bi, jnp.float32)
            y = y * (1.5 - 0.5 * d * y * y)
            y = y * (1.5 - 0.5 * d * y * y)
            y = y * (1.5 - 0.5 * d * y * y)
            b_chunk[pl.ds(i * 16, 16)] = y
            return c
        lax.fori_loop(0, NV_C, newton, 0)
        pltpu.sync_copy(b_chunk, sh_g.at[pl.ds(tid * CB, CH)])
    plsc.subcore_barrier()
    @pl.when(core == 0)
    def _():
        pltpu.sync_copy(sh_g, b_dinv)
    plsc.subcore_barrier()

    @pl.when(core == 0)
    def _():
        pltpu.sync_copy(g0_hbm, b_g)
        pltpu.sync_copy(x_hbm.at[pl.ds(tid * CB, CH)], b_x)

        @plsc.parallel_loop(0, NV_C, unroll=4)
        def _(i):
            xi = b_x[pl.ds(i * 16, 16)]
            d16 = b_dinv[pl.ds(tid * CB + i * 16, 16)]
            b_chunk[pl.ds(i * 16, 16)] = plsc.load_gather(b_g, [xi]) * d16
        pltpu.sync_copy(b_chunk, sh_g.at[pl.ds(tid * CB, CH)])
    plsc.subcore_barrier()

    def graph_round(s_off, write_hbm):
        @pl.when(core == 0)
        def _():
            pltpu.sync_copy(sh_g, b_g)
            zero_acc()

            def edge_step(i, c):
                sv = b_src[pl.ds(i * 16, 16)]
                dv = b_dst[pl.ds(i * 16, 16)]
                ps = plsc.load_gather(b_g, [sv])
                plsc.addupdate_scatter(b_acc, [dv], ps)
                return c
            lax.fori_loop(0, NV_E, edge_step, 0, unroll=16)
            pltpu.sync_copy(
                b_acc,
                sh_all.at[pl.ds(pl.multiple_of(tid * N_NODES, 8), N_NODES)])
        plsc.subcore_barrier()

        @pl.when(core == 0)
        def _():
            combine_my_chunk()
            sv = b_svec[pl.ds(s_off, 16)]

            @plsc.parallel_loop(0, NV_C, unroll=4)
            def _(i):
                p16 = b_g[pl.ds(tid * CB + i * 16, 16)]
                d16 = b_dinv[pl.ds(tid * CB + i * 16, 16)]
                g_new = (b_chunk[pl.ds(i * 16, 16)] + p16) * d16 + sv
                if write_hbm:
                    b_chunk[pl.ds(i * 16, 16)] = g_new
                else:
                    b_chunk[pl.ds(i * 16, 16)] = g_new * d16
            if write_hbm:
                pltpu.sync_copy(b_chunk, v_out.at[pl.ds(tid * CB, CH)])
            else:
                pltpu.sync_copy(b_chunk, sh_g.at[pl.ds(tid * CB, CH)])
        if not write_hbm:
            plsc.subcore_barrier()

    graph_round(0, False)
    graph_round(16, True)


def _sc_graph(src, dst, x, g0, svec, user_r, u_table):
    mesh = plsc.VectorSubcoreMesh(core_axis_name="c", subcore_axis_name="s")
    f = pl.kernel(
        _sc_body,
        out_type=[
            jax.ShapeDtypeStruct((N_NODES,), jnp.float32),
            jax.ShapeDtypeStruct((BATCH, DIM), jnp.float32),
        ],
        mesh=mesh,
        scratch_types=[
            pltpu.VMEM((EC,), jnp.int32),
            pltpu.VMEM((EC,), jnp.int32),
            pltpu.VMEM((N_NODES,), jnp.float32),
            pltpu.VMEM((N_NODES,), jnp.float32),
            pltpu.VMEM((N_NODES,), jnp.float32),
            pltpu.VMEM((CH,), jnp.int32),
            pltpu.VMEM((CH,), jnp.float32),
            pltpu.VMEM((CH,), jnp.float32),
            pltpu.VMEM((32,), jnp.float32),
            pltpu.VMEM((UB // 128, 128), jnp.int32),
            pltpu.VMEM((UB, DIM), jnp.float32),
            pltpu.VMEM_SHARED((NT * N_NODES,), jnp.float32),
            pltpu.VMEM_SHARED((N_NODES,), jnp.float32),
            pltpu.SemaphoreType.DMA,
        ],
        compiler_params=pltpu.CompilerParams(needs_layout_passes=False),
    )
    return f(src, dst, x, g0, svec, user_r, u_table)


_BN = 512
_GN = BATCH // _BN


def _final_body(itemT_ref, v_ref, uemb_ref, wu_ref, bl_ref, out_ref):
    vrow = v_ref[...].reshape(1, N_NODES)
    part = jnp.dot(vrow, itemT_ref[...], preferred_element_type=jnp.float32)
    su = lax.dot_general(wu_ref[...], uemb_ref[...],
                         (((1,), (1,)), ((), ())),
                         preferred_element_type=jnp.float32)
    out_ref[...] = jax.nn.sigmoid(part + su + bl_ref[0, 0])


def _final(itemT, v, uemb, wu2d, bl2d):
    return pl.pallas_call(
        _final_body,
        grid=(_GN,),
        in_specs=[
            pl.BlockSpec((N_NODES, _BN), lambda n: (0, n)),
            pl.BlockSpec((N_NODES,), lambda n: (0,)),
            pl.BlockSpec((_BN, DIM), lambda n: (n, 0)),
            pl.BlockSpec((1, DIM), lambda n: (0, 0)),
            pl.BlockSpec((1, 1), lambda n: (0, 0)),
        ],
        out_specs=pl.BlockSpec((1, _BN), lambda n: (0, n)),
        out_shape=jax.ShapeDtypeStruct((1, BATCH), jnp.float32),
        compiler_params=pltpu.CompilerParams(
            dimension_semantics=("arbitrary",),
            vmem_limit_bytes=50 * 1024 * 1024),
    )(itemT, v, uemb, wu2d, bl2d)


def kernel(user, item, x, edge_index, u_table, i_table, W1, b1, W2, b2,
           Wl, bl):
    WlT = Wl.reshape(1, 2 * DIM)
    g0, svec = _prep(i_table, W1, W2, WlT,
                     b1.reshape(1, DIM), b2.reshape(1, DIM))
    src = edge_index[0].astype(jnp.int32)
    dst = edge_index[1].astype(jnp.int32)
    user_r = user.astype(jnp.int32).reshape(NT, UB // 128, 128)
    v, uemb = _sc_graph(src, dst, x.astype(jnp.int32), g0, svec,
                        user_r, u_table)
    wu2d = WlT[:, :DIM]
    out_row = _final(item.T, v, uemb, wu2d, bl.reshape(1, 1))
    return out_row.reshape(BATCH, 1)


# speedup vs baseline: 1.6900x; 1.0012x over previous
import jax
import jax.numpy as jnp
from jax import lax
from jax.experimental import pallas as pl
from jax.experimental.pallas import tpu as pltpu
from jax.experimental.pallas import tpu_sc as plsc

N_NODES = 10000
N_EDGES = 320000
BATCH = 4096
DIM = 128

NT = 16
EC = N_EDGES // NT
CB = 624
CH = 640
NV_N = N_NODES // 16
NV_E = EC // 16
NV_C = CH // 16
UB = BATCH // NT


def _contract1(row, mat):
    return lax.dot_general(row, mat, (((1,), (1,)), ((), ())),
                           preferred_element_type=jnp.float32)


def _prep_body(itab_ref, w1_ref, w2_ref, wlT_ref, b1_ref, b2_ref,
               g0_ref, svec_ref):
    wiT = wlT_ref[:, 128:256]
    w2iT = _contract1(wiT, w2_ref[...])
    w12T = _contract1(w2iT, w1_ref[...])
    g0_ref[...] = _contract1(w12T, itab_ref[...]).reshape(N_NODES)
    s1 = jnp.sum(b1_ref[...] * w2iT)
    s2 = jnp.sum(b2_ref[...] * wiT)
    col = lax.broadcasted_iota(jnp.int32, (1, 32), 1)
    svec_ref[...] = jnp.where(col < 16, s1, s2).reshape(32)


def _prep(i_table, W1, W2, WlT, b1_2d, b2_2d):
    return pl.pallas_call(
        _prep_body,
        out_shape=[
            jax.ShapeDtypeStruct((N_NODES,), jnp.float32),
            jax.ShapeDtypeStruct((32,), jnp.float32),
        ],
    )(i_table, W1, W2, WlT, b1_2d, b2_2d)


def _sc_body(src_hbm, dst_hbm, x_hbm, g0_hbm, svec_hbm, user_hbm, utab_hbm,
             v_out, uemb_out,
             b_src, b_dst, b_g, b_dinv, b_acc, b_x, b_chunk, b_chunk2,
             b_svec, b_uidx, b_urows, sh_all, sh_g, sem):
    core = lax.axis_index("c")
    tid = lax.axis_index("s")

    @pl.when(core == 1)
    def _():
        pltpu.sync_copy(user_hbm.at[tid], b_uidx)
        for j in range(UB // 128):
            pltpu.async_copy(utab_hbm.at[b_uidx.at[j]],
                             b_urows.at[pl.ds(j * 128, 128)], sem).wait()
        pltpu.sync_copy(b_urows, uemb_out.at[pl.ds(tid * UB, UB)])

    def zero_acc():
        @plsc.parallel_loop(0, NV_N, unroll=8)
        def _(i):
            b_acc[pl.ds(i * 16, 16)] = jnp.zeros((16,), jnp.float32)

    def my_row_off(t):
        return pl.multiple_of(t * N_NODES + tid * CB, 8)

    def combine_my_chunk():
        pltpu.sync_copy(sh_all.at[pl.ds(my_row_off(0), CH)], b_chunk)
        for t in range(1, NT):
            pltpu.sync_copy(sh_all.at[pl.ds(my_row_off(t), CH)], b_chunk2)

            def addrow(i, c):
                b_chunk[pl.ds(i * 16, 16)] = (
                    b_chunk[pl.ds(i * 16, 16)] + b_chunk2[pl.ds(i * 16, 16)])
                return c
            lax.fori_loop(0, NV_C, addrow, 0)

    @pl.when(core == 0)
    def _():
        pltpu.sync_copy(src_hbm.at[pl.ds(tid * EC, EC)], b_src)
        pltpu.sync_copy(dst_hbm.at[pl.ds(tid * EC, EC)], b_dst)
        pltpu.sync_copy(svec_hbm, b_svec)
        zero_acc()
        ones = jnp.full((16,), 1.0, jnp.float32)

        def deg_step(i, c):
            dv = b_dst[pl.ds(i * 16, 16)]
            plsc.addupdate_scatter(b_acc, [dv], ones)
            return c
        lax.fori_loop(0, NV_E, deg_step, 0, unroll=16)
        pltpu.sync_copy(
            b_acc,
            sh_all.at[pl.ds(pl.multiple_of(tid * N_NODES, 8), N_NODES)])
    plsc.subcore_barrier()

    @pl.when(core == 0)
    def _():
        combine_my_chunk()

        def newton(i, c):
            d = b_chunk[pl.ds(i * 16, 16)] + 1.0
            bi = plsc.bitcast(d, jnp.int32)
            bi = 0x5F3759DF - lax.shift_right_arithmetic(bi, 1)
            y = plsc.bitcast(bi, jnp.float32)
            y = y * (1.5 - 0.5 * d * y * y)
            y = y * (1.5 - 0.5 * d * y * y)
            y = y * (1.5 - 0.5 * d * y * y)
            b_chunk[pl.ds(i * 16, 16)] = y
            return c
        lax.fori_loop(0, NV_C, newton, 0)
        pltpu.sync_copy(b_chunk, sh_g.at[pl.ds(tid * CB, CH)])
    plsc.subcore_barrier()
    @pl.when(core == 0)
    def _():
        pltpu.sync_copy(sh_g, b_dinv)
    plsc.subcore_barrier()

    @pl.when(core == 0)
    def _():
        pltpu.sync_copy(g0_hbm, b_g)
        pltpu.sync_copy(x_hbm.at[pl.ds(tid * CB, CH)], b_x)

        @plsc.parallel_loop(0, NV_C, unroll=4)
        def _(i):
            xi = b_x[pl.ds(i * 16, 16)]
            d16 = b_dinv[pl.ds(tid * CB + i * 16, 16)]
            b_chunk[pl.ds(i * 16, 16)] = plsc.load_gather(b_g, [xi]) * d16
        pltpu.sync_copy(b_chunk, sh_g.at[pl.ds(tid * CB, CH)])
    plsc.subcore_barrier()

    def graph_round(s_off, write_hbm):
        @pl.when(core == 0)
        def _():
            pltpu.sync_copy(sh_g, b_g)
            zero_acc()

            def edge_step(i, c):
                sv = b_src[pl.ds(i * 16, 16)]
                dv = b_dst[pl.ds(i * 16, 16)]
                ps = plsc.load_gather(b_g, [sv])
                plsc.addupdate_scatter(b_acc, [dv], ps)
                return c
            lax.fori_loop(0, NV_E, edge_step, 0, unroll=16)
            pltpu.sync_copy(
                b_acc,
                sh_all.at[pl.ds(pl.multiple_of(tid * N_NODES, 8), N_NODES)])
        plsc.subcore_barrier()

        @pl.when(core == 0)
        def _():
            combine_my_chunk()
            sv = b_svec[pl.ds(s_off, 16)]

            @plsc.parallel_loop(0, NV_C, unroll=4)
            def _(i):
                p16 = b_g[pl.ds(tid * CB + i * 16, 16)]
                d16 = b_dinv[pl.ds(tid * CB + i * 16, 16)]
                g_new = (b_chunk[pl.ds(i * 16, 16)] + p16) * d16 + sv
                if write_hbm:
                    b_chunk[pl.ds(i * 16, 16)] = g_new
                else:
                    b_chunk[pl.ds(i * 16, 16)] = g_new * d16
            if write_hbm:
                pltpu.sync_copy(b_chunk, v_out.at[pl.ds(tid * CB, CH)])
            else:
                pltpu.sync_copy(b_chunk, sh_g.at[pl.ds(tid * CB, CH)])
        if not write_hbm:
            plsc.subcore_barrier()

    graph_round(0, False)
    graph_round(16, True)


def _sc_graph(src, dst, x, g0, svec, user_r, u_table):
    mesh = plsc.VectorSubcoreMesh(core_axis_name="c", subcore_axis_name="s")
    f = pl.kernel(
        _sc_body,
        out_type=[
            jax.ShapeDtypeStruct((N_NODES,), jnp.float32),
            jax.ShapeDtypeStruct((BATCH, DIM), jnp.float32),
        ],
        mesh=mesh,
        scratch_types=[
            pltpu.VMEM((EC,), jnp.int32),
            pltpu.VMEM((EC,), jnp.int32),
            pltpu.VMEM((N_NODES,), jnp.float32),
            pltpu.VMEM((N_NODES,), jnp.float32),
            pltpu.VMEM((N_NODES,), jnp.float32),
            pltpu.VMEM((CH,), jnp.int32),
            pltpu.VMEM((CH,), jnp.float32),
            pltpu.VMEM((CH,), jnp.float32),
            pltpu.VMEM((32,), jnp.float32),
            pltpu.VMEM((UB // 128, 128), jnp.int32),
            pltpu.VMEM((UB, DIM), jnp.float32),
            pltpu.VMEM_SHARED((NT * N_NODES,), jnp.float32),
            pltpu.VMEM_SHARED((N_NODES,), jnp.float32),
            pltpu.SemaphoreType.DMA,
        ],
        compiler_params=pltpu.CompilerParams(needs_layout_passes=False),
    )
    return f(src, dst, x, g0, svec, user_r, u_table)


_BN = 512
_GN = BATCH // _BN


def _final_body(itemT_ref, v_ref, uemb_ref, wu_ref, bl_ref, out_ref):
    vrow = v_ref[...].reshape(1, N_NODES)
    part = jnp.dot(vrow, itemT_ref[...], preferred_element_type=jnp.float32)
    su = lax.dot_general(wu_ref[...], uemb_ref[...],
                         (((1,), (1,)), ((), ())),
                         preferred_element_type=jnp.float32)
    out_ref[...] = jax.nn.sigmoid(part + su + bl_ref[0, 0])


def _final(itemT, v, uemb, wu2d, bl2d):
    return pl.pallas_call(
        _final_body,
        grid=(_GN,),
        in_specs=[
            pl.BlockSpec((N_NODES, _BN), lambda n: (0, n)),
            pl.BlockSpec((N_NODES,), lambda n: (0,)),
            pl.BlockSpec((_BN, DIM), lambda n: (n, 0)),
            pl.BlockSpec((1, DIM), lambda n: (0, 0)),
            pl.BlockSpec((1, 1), lambda n: (0, 0)),
        ],
        out_specs=pl.BlockSpec((1, _BN), lambda n: (0, n)),
        out_shape=jax.ShapeDtypeStruct((1, BATCH), jnp.float32),
        compiler_params=pltpu.CompilerParams(
            dimension_semantics=("arbitrary",),
            vmem_limit_bytes=50 * 1024 * 1024),
    )(itemT, v, uemb, wu2d, bl2d)


def kernel(user, item, x, edge_index, u_table, i_table, W1, b1, W2, b2,
           Wl, bl):
    WlT = Wl.reshape(1, 2 * DIM)
    g0, svec = _prep(i_table, W1, W2, WlT,
                     b1.reshape(1, DIM), b2.reshape(1, DIM))
    src = edge_index[0].astype(jnp.int32)
    dst = edge_index[1].astype(jnp.int32)
    user_r = user.astype(jnp.int32).reshape(NT, UB // 128, 128)
    v, uemb = _sc_graph(src, dst, x.astype(jnp.int32), g0, svec,
                        user_r, u_table)
    wu2d = WlT[:, :DIM]
    out_row = _final(item.T, v, uemb, wu2d, bl.reshape(1, 1))
    return out_row.reshape(BATCH, 1)
